# Initial kernel scaffold; baseline (speedup 1.0000x reference)
import jax
import jax.numpy as jnp
from jax.experimental import pallas as pl


def kernel(x, edge_index, batch, W1_0, b1_0, gamma_0, beta_0, W2_0, b2_0, W1_1, b1_1, gamma_1, beta_1, W2_1, b2_1):
    raise NotImplementedError("write your pallas kernel here")


# speedup vs baseline: 4.2334x; 4.2334x over previous
import functools

import jax
import jax.numpy as jnp
from jax import lax
from jax.experimental import pallas as pl
from jax.experimental.pallas import tpu as pltpu
from jax.experimental.pallas import tpu_sc as plsc

NC = 2
NS = 16
NW = NC * NS

_EPS = 1e-5


def _sc_segment_sum(x, src, dst, zeros):
    n, d = x.shape
    e = src.shape[0]
    epw = e // NW
    chunk = 80
    steps = epw // chunk
    units = n // 8
    upt = units // NS
    rpt = upt * 8
    extras = units - upt * NS
    mesh = plsc.VectorSubcoreMesh(core_axis_name="c", subcore_axis_name="s")

    @functools.partial(
        pl.kernel,
        out_type=jax.ShapeDtypeStruct((NC, n, d), jnp.float32),
        mesh=mesh,
        scratch_types=[
            pltpu.VMEM((chunk,), jnp.int32),
            pltpu.VMEM((chunk,), jnp.int32),
            pltpu.VMEM((chunk, d), jnp.float32),
            pltpu.MemorySpace.VMEM_SHARED((n, d), jnp.float32),
            pltpu.SemaphoreType.DMA,
        ],
    )
    def k(x_hbm, src_hbm, dst_hbm, z_hbm, out_hbm, sidx, didx, rows, acc, sem):
        cid = lax.axis_index("c")
        sid = lax.axis_index("s")
        wid = sid * NC + cid
        rbase = pl.multiple_of(sid * rpt, 8)
        xbase = pl.multiple_of(upt * NS * 8 + sid * 8, 8)

        def init_from(ref):
            pltpu.sync_copy(ref.at[pl.ds(rbase, rpt)], acc.at[pl.ds(rbase, rpt)])

            @pl.when(sid < extras)
            def _():
                pltpu.sync_copy(ref.at[pl.ds(xbase, 8)], acc.at[pl.ds(xbase, 8)])

        @pl.when(cid == 0)
        def _():
            init_from(x_hbm)

        @pl.when(cid != 0)
        def _():
            init_from(z_hbm)

        plsc.subcore_barrier()

        ebase = wid * epw

        def body(i, carry):
            base = pl.multiple_of(ebase + i * chunk, 8)
            pltpu.sync_copy(src_hbm.at[pl.ds(base, chunk)], sidx)
            pltpu.async_copy(x_hbm.at[sidx], rows, sem).wait()
            pltpu.sync_copy(dst_hbm.at[pl.ds(base, chunk)], didx)
            pltpu.sync_copy(rows, acc.at[didx], add=True)
            return carry

        lax.fori_loop(0, steps, body, 0)
        plsc.subcore_barrier()
        pltpu.sync_copy(acc.at[pl.ds(rbase, rpt)],
                        out_hbm.at[cid].at[pl.ds(rbase, rpt)])

        @pl.when(sid < extras)
        def _():
            pltpu.sync_copy(acc.at[pl.ds(xbase, 8)],
                            out_hbm.at[cid].at[pl.ds(xbase, 8)])

    return k(x, src, dst, zeros)


_ROWS = 1000


def _stats_fold(agg0, agg1, w1, b1, gamma, beta):
    n, d = agg0.shape
    grid = n // _ROWS
    inv_n = 1.0 / n

    def body(a0, a1, w1r, b1r, gr, br, xinr, w1pr, b1pr, sacc, xacc):
        i = pl.program_id(0)
        xin = a0[...] + a1[...]
        xinr[...] = xin

        @pl.when(i == 0)
        def _():
            sacc[...] = jnp.zeros_like(sacc)
            xacc[...] = jnp.zeros_like(xacc)

        sacc[...] += jnp.sum(xin, axis=0, keepdims=True)
        xacc[...] += lax.dot_general(xin, xin, (((0,), (0,)), ((), ())),
                                     preferred_element_type=jnp.float32,
                  precision=lax.Precision.HIGHEST)

        @pl.when(i == pl.num_programs(0) - 1)
        def _():
            w = w1r[...]
            m = sacc[...] * inv_n
            mw = jnp.dot(m, w, preferred_element_type=jnp.float32,
                  precision=lax.Precision.HIGHEST)
            xw = jnp.dot(xacc[...], w, preferred_element_type=jnp.float32,
                  precision=lax.Precision.HIGHEST)
            q = jnp.sum(w * xw, axis=0, keepdims=True) * inv_n
            var = q - mw * mw
            scale = gr[...] * lax.rsqrt(var + _EPS)
            w1pr[...] = w * scale
            b1pr[...] = br[...] - mw * scale

    return pl.pallas_call(
        body,
        grid=(grid,),
        in_specs=[
            pl.BlockSpec((_ROWS, d), lambda i: (i, 0)),
            pl.BlockSpec((_ROWS, d), lambda i: (i, 0)),
            pl.BlockSpec((d, d), lambda i: (0, 0)),
            pl.BlockSpec((1, d), lambda i: (0, 0)),
            pl.BlockSpec((1, d), lambda i: (0, 0)),
            pl.BlockSpec((1, d), lambda i: (0, 0)),
        ],
        out_specs=[
            pl.BlockSpec((_ROWS, d), lambda i: (i, 0)),
            pl.BlockSpec((d, d), lambda i: (0, 0)),
            pl.BlockSpec((1, d), lambda i: (0, 0)),
        ],
        out_shape=[
            jax.ShapeDtypeStruct((n, d), jnp.float32),
            jax.ShapeDtypeStruct((d, d), jnp.float32),
            jax.ShapeDtypeStruct((1, d), jnp.float32),
        ],
        scratch_shapes=[
            pltpu.VMEM((1, d), jnp.float32),
            pltpu.VMEM((d, d), jnp.float32),
        ],
    )(agg0, agg1, w1, b1, gamma, beta)


def _mlp(xin, w1p, b1p, w2, b2):
    n, d = xin.shape
    grid = n // _ROWS

    def body(xr, w1r, b1r, w2r, b2r, hr):
        h1 = jnp.maximum(
            jnp.dot(xr[...], w1r[...], preferred_element_type=jnp.float32,
                  precision=lax.Precision.HIGHEST)
            + b1r[...], 0.0)
        hr[...] = jnp.maximum(
            jnp.dot(h1, w2r[...], preferred_element_type=jnp.float32,
                  precision=lax.Precision.HIGHEST)
            + b2r[...], 0.0)

    return pl.pallas_call(
        body,
        grid=(grid,),
        in_specs=[
            pl.BlockSpec((_ROWS, d), lambda i: (i, 0)),
            pl.BlockSpec((d, d), lambda i: (0, 0)),
            pl.BlockSpec((1, d), lambda i: (0, 0)),
            pl.BlockSpec((d, d), lambda i: (0, 0)),
            pl.BlockSpec((1, d), lambda i: (0, 0)),
        ],
        out_specs=pl.BlockSpec((_ROWS, d), lambda i: (i, 0)),
        out_shape=jax.ShapeDtypeStruct((n, d), jnp.float32),
    )(xin, w1p, b1p, w2, b2)


def _mlp_pool(xin, w1p, b1p, w2, b2, bid, g):
    n, d = xin.shape
    grid = n // _ROWS

    def body(xr, w1r, b1r, w2r, b2r, br_ids, outr, sacc, cacc):
        i = pl.program_id(0)
        h1 = jnp.maximum(
            jnp.dot(xr[...], w1r[...], preferred_element_type=jnp.float32,
                  precision=lax.Precision.HIGHEST)
            + b1r[...], 0.0)
        h2 = jnp.maximum(
            jnp.dot(h1, w2r[...], preferred_element_type=jnp.float32,
                  precision=lax.Precision.HIGHEST)
            + b2r[...], 0.0)
        ids = br_ids[...]
        gcol = lax.broadcasted_iota(jnp.int32, (_ROWS, g), 1).astype(jnp.float32)
        onehot = (ids == gcol).astype(jnp.float32)

        @pl.when(i == 0)
        def _():
            sacc[...] = jnp.zeros_like(sacc)
            cacc[...] = jnp.zeros_like(cacc)

        sacc[...] += lax.dot_general(onehot, h2, (((0,), (0,)), ((), ())),
                                     preferred_element_type=jnp.float32,
                  precision=lax.Precision.HIGHEST)
        cacc[...] += lax.dot_general(onehot, jnp.ones_like(h2),
                                     (((0,), (0,)), ((), ())),
                                     preferred_element_type=jnp.float32,
                  precision=lax.Precision.HIGHEST)

        @pl.when(i == pl.num_programs(0) - 1)
        def _():
            outr[...] = sacc[...] / jnp.maximum(cacc[...], 1.0)

    return pl.pallas_call(
        body,
        grid=(grid,),
        in_specs=[
            pl.BlockSpec((_ROWS, d), lambda i: (i, 0)),
            pl.BlockSpec((d, d), lambda i: (0, 0)),
            pl.BlockSpec((1, d), lambda i: (0, 0)),
            pl.BlockSpec((d, d), lambda i: (0, 0)),
            pl.BlockSpec((1, d), lambda i: (0, 0)),
            pl.BlockSpec((_ROWS, 1), lambda i: (i, 0)),
        ],
        out_specs=pl.BlockSpec((g, d), lambda i: (0, 0)),
        out_shape=jax.ShapeDtypeStruct((g, d), jnp.float32),
        scratch_shapes=[
            pltpu.VMEM((g, d), jnp.float32),
            pltpu.VMEM((g, d), jnp.float32),
        ],
    )(xin, w1p, b1p, w2, b2, bid)


def kernel(x, edge_index, batch, W1_0, b1_0, gamma_0, beta_0, W2_0, b2_0,
           W1_1, b1_1, gamma_1, beta_1, W2_1, b2_1):
    n, d = x.shape
    g = 64
    src = edge_index[0]
    dst = edge_index[1]
    z = jnp.zeros_like(x)
    bid = batch.astype(jnp.float32).reshape(n, 1)

    agg = _sc_segment_sum(x, src, dst, z)
    xin, w1p, b1p = _stats_fold(agg[0], agg[1], W1_0, b1_0.reshape(1, d),
                                gamma_0.reshape(1, d), beta_0.reshape(1, d))
    h = _mlp(xin, w1p, b1p, W2_0, b2_0.reshape(1, d))

    agg2 = _sc_segment_sum(h, src, dst, z)
    xin2, w1p2, b1p2 = _stats_fold(agg2[0], agg2[1], W1_1, b1_1.reshape(1, d),
                                   gamma_1.reshape(1, d), beta_1.reshape(1, d))
    out = _mlp_pool(xin2, w1p2, b1p2, W2_1, b2_1.reshape(1, d), bid, g)
    return out


# speedup vs baseline: 7.3065x; 1.7259x over previous
import functools

import jax
import jax.numpy as jnp
from jax import lax
from jax.experimental import pallas as pl
from jax.experimental.pallas import tpu as pltpu
from jax.experimental.pallas import tpu_sc as plsc

NC = 2
NS = 16
NW = NC * NS

_EPS = 1e-5


_CHUNK = 100
_SLOTS = 2


def _sc_segment_sum(x, src3, dst3, zeros):
    n, d = x.shape
    _, steps, chunk = src3.shape
    units = n // 8
    upt = units // NS
    rpt = upt * 8
    extras = units - upt * NS
    mesh = plsc.VectorSubcoreMesh(core_axis_name="c", subcore_axis_name="s")

    @functools.partial(
        pl.kernel,
        out_type=jax.ShapeDtypeStruct((NC, n, d), jnp.float32),
        mesh=mesh,
        scratch_types=[
            pltpu.VMEM((steps, chunk), jnp.int32),
            [pltpu.VMEM((chunk,), jnp.int32)] * _SLOTS,
            [pltpu.VMEM((chunk, d), jnp.float32)] * _SLOTS,
            pltpu.MemorySpace.VMEM_SHARED((n, d), jnp.float32),
            [pltpu.SemaphoreType.DMA] * _SLOTS,
            [pltpu.SemaphoreType.DMA] * _SLOTS,
            [pltpu.SemaphoreType.DMA] * _SLOTS,
        ],
    )
    def k(x_hbm, src_hbm, dst_hbm, z_hbm, out_hbm, sidx, didx, rows, acc,
          gsem, dsem, ssem):
        cid = lax.axis_index("c")
        sid = lax.axis_index("s")
        wid = sid * NC + cid
        rbase = pl.multiple_of(sid * rpt, 8)
        xbase = pl.multiple_of(upt * NS * 8 + sid * 8, 8)

        def init_from(ref):
            pltpu.sync_copy(ref.at[pl.ds(rbase, rpt)], acc.at[pl.ds(rbase, rpt)])

            @pl.when(sid < extras)
            def _():
                pltpu.sync_copy(ref.at[pl.ds(xbase, 8)], acc.at[pl.ds(xbase, 8)])

        @pl.when(cid == 0)
        def _():
            init_from(x_hbm)

        @pl.when(cid != 0)
        def _():
            init_from(z_hbm)

        pltpu.sync_copy(src_hbm.at[wid], sidx)
        plsc.subcore_barrier()

        def start_fetch(j, slot):
            pltpu.async_copy(dst_hbm.at[wid].at[j], didx[slot], dsem[slot])
            pltpu.async_copy(x_hbm.at[sidx.at[j]], rows[slot], gsem[slot])

        def wait_fetch(slot):
            pltpu.make_async_copy(dst_hbm.at[wid].at[0], didx[slot],
                                  dsem[slot]).wait()
            pltpu.make_async_copy(x_hbm.at[sidx.at[0]], rows[slot],
                                  gsem[slot]).wait()

        for s in range(_SLOTS):
            start_fetch(s, s)

        def body(i, carry):
            j = i * _SLOTS
            scat = []
            for s in range(_SLOTS):
                wait_fetch(s)
                scat.append(pltpu.async_copy(rows[s], acc.at[didx[s]],
                                             ssem[s], add=True))
            for s in range(_SLOTS):
                scat[s].wait()
                start_fetch(j + s + _SLOTS, s)
            return carry

        lax.fori_loop(0, steps // _SLOTS - 1, body, 0)
        for s in range(_SLOTS):
            wait_fetch(s)
            pltpu.async_copy(rows[s], acc.at[didx[s]],
                             ssem[s], add=True).wait()
        plsc.subcore_barrier()
        pltpu.sync_copy(acc.at[pl.ds(rbase, rpt)],
                        out_hbm.at[cid].at[pl.ds(rbase, rpt)])

        @pl.when(sid < extras)
        def _():
            pltpu.sync_copy(acc.at[pl.ds(xbase, 8)],
                            out_hbm.at[cid].at[pl.ds(xbase, 8)])

    return k(x, src3, dst3, zeros)


_ROWS = 1000


def _stats_fold(agg0, agg1, w1, b1, gamma, beta):
    n, d = agg0.shape
    grid = n // _ROWS
    inv_n = 1.0 / n

    def body(a0, a1, w1r, b1r, gr, br, xinr, w1pr, b1pr, sacc, xacc):
        i = pl.program_id(0)
        xin = a0[...] + a1[...]
        xinr[...] = xin

        @pl.when(i == 0)
        def _():
            sacc[...] = jnp.zeros_like(sacc)
            xacc[...] = jnp.zeros_like(xacc)

        sacc[...] += jnp.sum(xin, axis=0, keepdims=True)
        xacc[...] += lax.dot_general(xin, xin, (((0,), (0,)), ((), ())),
                                     preferred_element_type=jnp.float32,
                  precision=lax.Precision.HIGHEST)

        @pl.when(i == pl.num_programs(0) - 1)
        def _():
            w = w1r[...]
            m = sacc[...] * inv_n
            mw = jnp.dot(m, w, preferred_element_type=jnp.float32,
                  precision=lax.Precision.HIGHEST)
            xw = jnp.dot(xacc[...], w, preferred_element_type=jnp.float32,
                  precision=lax.Precision.HIGHEST)
            q = jnp.sum(w * xw, axis=0, keepdims=True) * inv_n
            var = q - mw * mw
            scale = gr[...] * lax.rsqrt(var + _EPS)
            w1pr[...] = w * scale
            b1pr[...] = br[...] - mw * scale

    return pl.pallas_call(
        body,
        grid=(grid,),
        in_specs=[
            pl.BlockSpec((_ROWS, d), lambda i: (i, 0)),
            pl.BlockSpec((_ROWS, d), lambda i: (i, 0)),
            pl.BlockSpec((d, d), lambda i: (0, 0)),
            pl.BlockSpec((1, d), lambda i: (0, 0)),
            pl.BlockSpec((1, d), lambda i: (0, 0)),
            pl.BlockSpec((1, d), lambda i: (0, 0)),
        ],
        out_specs=[
            pl.BlockSpec((_ROWS, d), lambda i: (i, 0)),
            pl.BlockSpec((d, d), lambda i: (0, 0)),
            pl.BlockSpec((1, d), lambda i: (0, 0)),
        ],
        out_shape=[
            jax.ShapeDtypeStruct((n, d), jnp.float32),
            jax.ShapeDtypeStruct((d, d), jnp.float32),
            jax.ShapeDtypeStruct((1, d), jnp.float32),
        ],
        scratch_shapes=[
            pltpu.VMEM((1, d), jnp.float32),
            pltpu.VMEM((d, d), jnp.float32),
        ],
    )(agg0, agg1, w1, b1, gamma, beta)


def _mlp(xin, w1p, b1p, w2, b2):
    n, d = xin.shape
    grid = n // _ROWS

    def body(xr, w1r, b1r, w2r, b2r, hr):
        h1 = jnp.maximum(
            jnp.dot(xr[...], w1r[...], preferred_element_type=jnp.float32,
                  precision=lax.Precision.HIGHEST)
            + b1r[...], 0.0)
        hr[...] = jnp.maximum(
            jnp.dot(h1, w2r[...], preferred_element_type=jnp.float32,
                  precision=lax.Precision.HIGHEST)
            + b2r[...], 0.0)

    return pl.pallas_call(
        body,
        grid=(grid,),
        in_specs=[
            pl.BlockSpec((_ROWS, d), lambda i: (i, 0)),
            pl.BlockSpec((d, d), lambda i: (0, 0)),
            pl.BlockSpec((1, d), lambda i: (0, 0)),
            pl.BlockSpec((d, d), lambda i: (0, 0)),
            pl.BlockSpec((1, d), lambda i: (0, 0)),
        ],
        out_specs=pl.BlockSpec((_ROWS, d), lambda i: (i, 0)),
        out_shape=jax.ShapeDtypeStruct((n, d), jnp.float32),
    )(xin, w1p, b1p, w2, b2)


def _mlp_pool(xin, w1p, b1p, w2, b2, bid, g):
    n, d = xin.shape
    grid = n // _ROWS

    def body(xr, w1r, b1r, w2r, b2r, br_ids, outr, sacc, cacc):
        i = pl.program_id(0)
        h1 = jnp.maximum(
            jnp.dot(xr[...], w1r[...], preferred_element_type=jnp.float32,
                  precision=lax.Precision.HIGHEST)
            + b1r[...], 0.0)
        h2 = jnp.maximum(
            jnp.dot(h1, w2r[...], preferred_element_type=jnp.float32,
                  precision=lax.Precision.HIGHEST)
            + b2r[...], 0.0)
        ids = br_ids[...]
        gcol = lax.broadcasted_iota(jnp.int32, (_ROWS, g), 1).astype(jnp.float32)
        onehot = (ids == gcol).astype(jnp.float32)

        @pl.when(i == 0)
        def _():
            sacc[...] = jnp.zeros_like(sacc)
            cacc[...] = jnp.zeros_like(cacc)

        sacc[...] += lax.dot_general(onehot, h2, (((0,), (0,)), ((), ())),
                                     preferred_element_type=jnp.float32,
                  precision=lax.Precision.HIGHEST)
        cacc[...] += lax.dot_general(onehot, jnp.ones_like(h2),
                                     (((0,), (0,)), ((), ())),
                                     preferred_element_type=jnp.float32,
                  precision=lax.Precision.HIGHEST)

        @pl.when(i == pl.num_programs(0) - 1)
        def _():
            outr[...] = sacc[...] / jnp.maximum(cacc[...], 1.0)

    return pl.pallas_call(
        body,
        grid=(grid,),
        in_specs=[
            pl.BlockSpec((_ROWS, d), lambda i: (i, 0)),
            pl.BlockSpec((d, d), lambda i: (0, 0)),
            pl.BlockSpec((1, d), lambda i: (0, 0)),
            pl.BlockSpec((d, d), lambda i: (0, 0)),
            pl.BlockSpec((1, d), lambda i: (0, 0)),
            pl.BlockSpec((_ROWS, 1), lambda i: (i, 0)),
        ],
        out_specs=pl.BlockSpec((g, d), lambda i: (0, 0)),
        out_shape=jax.ShapeDtypeStruct((g, d), jnp.float32),
        scratch_shapes=[
            pltpu.VMEM((g, d), jnp.float32),
            pltpu.VMEM((g, d), jnp.float32),
        ],
    )(xin, w1p, b1p, w2, b2, bid)


def kernel(x, edge_index, batch, W1_0, b1_0, gamma_0, beta_0, W2_0, b2_0,
           W1_1, b1_1, gamma_1, beta_1, W2_1, b2_1):
    n, d = x.shape
    g = 64
    src3 = edge_index[0].reshape(NW, -1, _CHUNK)
    dst3 = edge_index[1].reshape(NW, -1, _CHUNK)
    z = jnp.zeros_like(x)
    bid = batch.astype(jnp.float32).reshape(n, 1)

    agg = _sc_segment_sum(x, src3, dst3, z)
    xin, w1p, b1p = _stats_fold(agg[0], agg[1], W1_0, b1_0.reshape(1, d),
                                gamma_0.reshape(1, d), beta_0.reshape(1, d))
    h = _mlp(xin, w1p, b1p, W2_0, b2_0.reshape(1, d))

    agg2 = _sc_segment_sum(h, src3, dst3, z)
    xin2, w1p2, b1p2 = _stats_fold(agg2[0], agg2[1], W1_1, b1_1.reshape(1, d),
                                   gamma_1.reshape(1, d), beta_1.reshape(1, d))
    out = _mlp_pool(xin2, w1p2, b1p2, W2_1, b2_1.reshape(1, d), bid, g)
    return out


# speedup vs baseline: 7.5180x; 1.0289x over previous
import functools

import jax
import jax.numpy as jnp
from jax import lax
from jax.experimental import pallas as pl
from jax.experimental.pallas import tpu as pltpu
from jax.experimental.pallas import tpu_sc as plsc

NC = 2
NS = 16
NW = NC * NS

_EPS = 1e-5


_CHUNK = 50
_SLOTS = 5


def _sc_segment_sum(x, src3, dst3, zeros):
    n, d = x.shape
    _, steps, chunk = dst3.shape
    units = n // 8
    upt = units // NS
    rpt = upt * 8
    extras = units - upt * NS
    mesh = plsc.VectorSubcoreMesh(core_axis_name="c", subcore_axis_name="s")

    @functools.partial(
        pl.kernel,
        out_type=jax.ShapeDtypeStruct((NC, n, d), jnp.float32),
        mesh=mesh,
        scratch_types=[
            [pltpu.VMEM((chunk,), jnp.int32)] * _SLOTS,
            [pltpu.VMEM((chunk,), jnp.int32)] * _SLOTS,
            [pltpu.VMEM((chunk, d), jnp.float32)] * _SLOTS,
            pltpu.MemorySpace.VMEM_SHARED((n, d), jnp.float32),
            [pltpu.SemaphoreType.DMA] * _SLOTS,
            [pltpu.SemaphoreType.DMA] * _SLOTS,
            [pltpu.SemaphoreType.DMA] * _SLOTS,
            [pltpu.SemaphoreType.DMA] * _SLOTS,
        ],
    )
    def k(x_hbm, src_hbm, dst_hbm, z_hbm, out_hbm, sidx, didx, rows, acc,
          isem, dsem, gsem, ssem):
        cid = lax.axis_index("c")
        sid = lax.axis_index("s")
        wid = sid * NC + cid
        rbase = pl.multiple_of(sid * rpt, 8)
        xbase = pl.multiple_of(upt * NS * 8 + sid * 8, 8)

        def init_from(ref):
            pltpu.sync_copy(ref.at[pl.ds(rbase, rpt)], acc.at[pl.ds(rbase, rpt)])

            @pl.when(sid < extras)
            def _():
                pltpu.sync_copy(ref.at[pl.ds(xbase, 8)], acc.at[pl.ds(xbase, 8)])

        @pl.when(cid == 0)
        def _():
            init_from(x_hbm)

        @pl.when(cid != 0)
        def _():
            init_from(z_hbm)

        def start_idx(j, slot):
            pltpu.async_copy(src_hbm.at[wid].at[j], sidx[slot], isem[slot])
            pltpu.async_copy(dst_hbm.at[wid].at[j], didx[slot], dsem[slot])

        def wait_idx(slot):
            pltpu.make_async_copy(src_hbm.at[wid].at[0], sidx[slot],
                                  isem[slot]).wait()
            pltpu.make_async_copy(dst_hbm.at[wid].at[0], didx[slot],
                                  dsem[slot]).wait()

        def wait_gather(slot):
            pltpu.make_async_copy(x_hbm.at[sidx[0]], rows[slot],
                                  gsem[slot]).wait()

        for s in range(_SLOTS):
            start_idx(s, s)

        def body(i, carry):
            j = i * _SLOTS
            scat = []
            for s in range(_SLOTS):
                wait_idx(s)
                pltpu.async_copy(x_hbm.at[sidx[s]], rows[s], gsem[s])
            for s in range(_SLOTS):
                wait_gather(s)
                scat.append(pltpu.async_copy(rows[s], acc.at[didx[s]],
                                             ssem[s], add=True))
            for s in range(_SLOTS):
                scat[s].wait()
                start_idx(j + s + _SLOTS, s)
            return carry

        lax.fori_loop(0, steps // _SLOTS - 1, body, 0)
        for s in range(_SLOTS):
            wait_idx(s)
            pltpu.async_copy(x_hbm.at[sidx[s]], rows[s], gsem[s])
        for s in range(_SLOTS):
            wait_gather(s)
            pltpu.async_copy(rows[s], acc.at[didx[s]],
                             ssem[s], add=True).wait()
        plsc.subcore_barrier()
        pltpu.sync_copy(acc.at[pl.ds(rbase, rpt)],
                        out_hbm.at[cid].at[pl.ds(rbase, rpt)])

        @pl.when(sid < extras)
        def _():
            pltpu.sync_copy(acc.at[pl.ds(xbase, 8)],
                            out_hbm.at[cid].at[pl.ds(xbase, 8)])

    return k(x, src3, dst3, zeros)


_ROWS = 1000


def _stats_fold(agg0, agg1, w1, b1, gamma, beta):
    n, d = agg0.shape
    grid = n // _ROWS
    inv_n = 1.0 / n

    def body(a0, a1, w1r, b1r, gr, br, xinr, w1pr, b1pr, sacc, xacc):
        i = pl.program_id(0)
        xin = a0[...] + a1[...]
        xinr[...] = xin

        @pl.when(i == 0)
        def _():
            sacc[...] = jnp.zeros_like(sacc)
            xacc[...] = jnp.zeros_like(xacc)

        sacc[...] += jnp.sum(xin, axis=0, keepdims=True)
        xacc[...] += lax.dot_general(xin, xin, (((0,), (0,)), ((), ())),
                                     preferred_element_type=jnp.float32,
                  precision=lax.Precision.HIGHEST)

        @pl.when(i == pl.num_programs(0) - 1)
        def _():
            w = w1r[...]
            m = sacc[...] * inv_n
            mw = jnp.dot(m, w, preferred_element_type=jnp.float32,
                  precision=lax.Precision.HIGHEST)
            xw = jnp.dot(xacc[...], w, preferred_element_type=jnp.float32,
                  precision=lax.Precision.HIGHEST)
            q = jnp.sum(w * xw, axis=0, keepdims=True) * inv_n
            var = q - mw * mw
            scale = gr[...] * lax.rsqrt(var + _EPS)
            w1pr[...] = w * scale
            b1pr[...] = br[...] - mw * scale

    return pl.pallas_call(
        body,
        grid=(grid,),
        in_specs=[
            pl.BlockSpec((_ROWS, d), lambda i: (i, 0)),
            pl.BlockSpec((_ROWS, d), lambda i: (i, 0)),
            pl.BlockSpec((d, d), lambda i: (0, 0)),
            pl.BlockSpec((1, d), lambda i: (0, 0)),
            pl.BlockSpec((1, d), lambda i: (0, 0)),
            pl.BlockSpec((1, d), lambda i: (0, 0)),
        ],
        out_specs=[
            pl.BlockSpec((_ROWS, d), lambda i: (i, 0)),
            pl.BlockSpec((d, d), lambda i: (0, 0)),
            pl.BlockSpec((1, d), lambda i: (0, 0)),
        ],
        out_shape=[
            jax.ShapeDtypeStruct((n, d), jnp.float32),
            jax.ShapeDtypeStruct((d, d), jnp.float32),
            jax.ShapeDtypeStruct((1, d), jnp.float32),
        ],
        scratch_shapes=[
            pltpu.VMEM((1, d), jnp.float32),
            pltpu.VMEM((d, d), jnp.float32),
        ],
    )(agg0, agg1, w1, b1, gamma, beta)


def _mlp(xin, w1p, b1p, w2, b2):
    n, d = xin.shape
    grid = n // _ROWS

    def body(xr, w1r, b1r, w2r, b2r, hr):
        h1 = jnp.maximum(
            jnp.dot(xr[...], w1r[...], preferred_element_type=jnp.float32,
                  precision=lax.Precision.HIGHEST)
            + b1r[...], 0.0)
        hr[...] = jnp.maximum(
            jnp.dot(h1, w2r[...], preferred_element_type=jnp.float32,
                  precision=lax.Precision.HIGHEST)
            + b2r[...], 0.0)

    return pl.pallas_call(
        body,
        grid=(grid,),
        in_specs=[
            pl.BlockSpec((_ROWS, d), lambda i: (i, 0)),
            pl.BlockSpec((d, d), lambda i: (0, 0)),
            pl.BlockSpec((1, d), lambda i: (0, 0)),
            pl.BlockSpec((d, d), lambda i: (0, 0)),
            pl.BlockSpec((1, d), lambda i: (0, 0)),
        ],
        out_specs=pl.BlockSpec((_ROWS, d), lambda i: (i, 0)),
        out_shape=jax.ShapeDtypeStruct((n, d), jnp.float32),
    )(xin, w1p, b1p, w2, b2)


def _mlp_pool(xin, w1p, b1p, w2, b2, bid, g):
    n, d = xin.shape
    grid = n // _ROWS

    def body(xr, w1r, b1r, w2r, b2r, br_ids, outr, sacc, cacc):
        i = pl.program_id(0)
        h1 = jnp.maximum(
            jnp.dot(xr[...], w1r[...], preferred_element_type=jnp.float32,
                  precision=lax.Precision.HIGHEST)
            + b1r[...], 0.0)
        h2 = jnp.maximum(
            jnp.dot(h1, w2r[...], preferred_element_type=jnp.float32,
                  precision=lax.Precision.HIGHEST)
            + b2r[...], 0.0)
        ids = br_ids[...]
        gcol = lax.broadcasted_iota(jnp.int32, (_ROWS, g), 1).astype(jnp.float32)
        onehot = (ids == gcol).astype(jnp.float32)

        @pl.when(i == 0)
        def _():
            sacc[...] = jnp.zeros_like(sacc)
            cacc[...] = jnp.zeros_like(cacc)

        sacc[...] += lax.dot_general(onehot, h2, (((0,), (0,)), ((), ())),
                                     preferred_element_type=jnp.float32,
                  precision=lax.Precision.HIGHEST)
        cacc[...] += lax.dot_general(onehot, jnp.ones_like(h2),
                                     (((0,), (0,)), ((), ())),
                                     preferred_element_type=jnp.float32,
                  precision=lax.Precision.HIGHEST)

        @pl.when(i == pl.num_programs(0) - 1)
        def _():
            outr[...] = sacc[...] / jnp.maximum(cacc[...], 1.0)

    return pl.pallas_call(
        body,
        grid=(grid,),
        in_specs=[
            pl.BlockSpec((_ROWS, d), lambda i: (i, 0)),
            pl.BlockSpec((d, d), lambda i: (0, 0)),
            pl.BlockSpec((1, d), lambda i: (0, 0)),
            pl.BlockSpec((d, d), lambda i: (0, 0)),
            pl.BlockSpec((1, d), lambda i: (0, 0)),
            pl.BlockSpec((_ROWS, 1), lambda i: (i, 0)),
        ],
        out_specs=pl.BlockSpec((g, d), lambda i: (0, 0)),
        out_shape=jax.ShapeDtypeStruct((g, d), jnp.float32),
        scratch_shapes=[
            pltpu.VMEM((g, d), jnp.float32),
            pltpu.VMEM((g, d), jnp.float32),
        ],
    )(xin, w1p, b1p, w2, b2, bid)


def kernel(x, edge_index, batch, W1_0, b1_0, gamma_0, beta_0, W2_0, b2_0,
           W1_1, b1_1, gamma_1, beta_1, W2_1, b2_1):
    n, d = x.shape
    g = 64
    src3 = edge_index[0].reshape(NW, -1, _CHUNK)
    dst3 = edge_index[1].reshape(NW, -1, _CHUNK)
    z = jnp.zeros_like(x)
    bid = batch.astype(jnp.float32).reshape(n, 1)

    agg = _sc_segment_sum(x, src3, dst3, z)
    xin, w1p, b1p = _stats_fold(agg[0], agg[1], W1_0, b1_0.reshape(1, d),
                                gamma_0.reshape(1, d), beta_0.reshape(1, d))
    h = _mlp(xin, w1p, b1p, W2_0, b2_0.reshape(1, d))

    agg2 = _sc_segment_sum(h, src3, dst3, z)
    xin2, w1p2, b1p2 = _stats_fold(agg2[0], agg2[1], W1_1, b1_1.reshape(1, d),
                                   gamma_1.reshape(1, d), beta_1.reshape(1, d))
    out = _mlp_pool(xin2, w1p2, b1p2, W2_1, b2_1.reshape(1, d), bid, g)
    return out


# speedup vs baseline: 8.1761x; 1.0875x over previous
import functools

import jax
import jax.numpy as jnp
from jax import lax
from jax.experimental import pallas as pl
from jax.experimental.pallas import tpu as pltpu
from jax.experimental.pallas import tpu_sc as plsc

NC = 2
NS = 16
NW = NC * NS

_EPS = 1e-5


_CHUNK = 50
_SLOTS = 5


def _sc_segment_sum(x, src3, dst3, zeros):
    n, d = x.shape
    _, steps, chunk = dst3.shape
    units = n // 8
    upt = units // NS
    rpt = upt * 8
    extras = units - upt * NS
    mesh = plsc.VectorSubcoreMesh(core_axis_name="c", subcore_axis_name="s")

    @functools.partial(
        pl.kernel,
        out_type=jax.ShapeDtypeStruct((NC, n, d), jnp.float32),
        mesh=mesh,
        scratch_types=[
            [pltpu.VMEM((chunk,), jnp.int32)] * _SLOTS,
            [pltpu.VMEM((chunk,), jnp.int32)] * _SLOTS,
            [pltpu.VMEM((chunk, d), jnp.float32)] * _SLOTS,
            pltpu.MemorySpace.VMEM_SHARED((n, d), jnp.float32),
            [pltpu.SemaphoreType.DMA] * _SLOTS,
            [pltpu.SemaphoreType.DMA] * _SLOTS,
            [pltpu.SemaphoreType.DMA] * _SLOTS,
            [pltpu.SemaphoreType.DMA] * _SLOTS,
        ],
    )
    def k(x_hbm, src_hbm, dst_hbm, z_hbm, out_hbm, sidx, didx, rows, acc,
          isem, dsem, gsem, ssem):
        cid = lax.axis_index("c")
        sid = lax.axis_index("s")
        wid = sid * NC + cid
        rbase = pl.multiple_of(sid * rpt, 8)
        xbase = pl.multiple_of(upt * NS * 8 + sid * 8, 8)

        def init_from(ref):
            pltpu.sync_copy(ref.at[pl.ds(rbase, rpt)], acc.at[pl.ds(rbase, rpt)])

            @pl.when(sid < extras)
            def _():
                pltpu.sync_copy(ref.at[pl.ds(xbase, 8)], acc.at[pl.ds(xbase, 8)])

        @pl.when(cid == 0)
        def _():
            init_from(x_hbm)

        @pl.when(cid != 0)
        def _():
            init_from(z_hbm)

        def start_idx(j, slot):
            pltpu.async_copy(src_hbm.at[wid].at[j], sidx[slot], isem[slot])
            pltpu.async_copy(dst_hbm.at[wid].at[j], didx[slot], dsem[slot])

        def wait_idx(slot):
            pltpu.make_async_copy(src_hbm.at[wid].at[0], sidx[slot],
                                  isem[slot]).wait()
            pltpu.make_async_copy(dst_hbm.at[wid].at[0], didx[slot],
                                  dsem[slot]).wait()

        def wait_gather(slot):
            pltpu.make_async_copy(x_hbm.at[sidx[0]], rows[slot],
                                  gsem[slot]).wait()

        for s in range(_SLOTS):
            start_idx(s, s)

        def body(i, carry):
            j = i * _SLOTS
            scat = []
            for s in range(_SLOTS):
                wait_idx(s)
                pltpu.async_copy(x_hbm.at[sidx[s]], rows[s], gsem[s])
            for s in range(_SLOTS):
                wait_gather(s)
                scat.append(pltpu.async_copy(rows[s], acc.at[didx[s]],
                                             ssem[s], add=True))
            for s in range(_SLOTS):
                scat[s].wait()
                start_idx(j + s + _SLOTS, s)
            return carry

        lax.fori_loop(0, steps // _SLOTS - 1, body, 0)
        for s in range(_SLOTS):
            wait_idx(s)
            pltpu.async_copy(x_hbm.at[sidx[s]], rows[s], gsem[s])
        for s in range(_SLOTS):
            wait_gather(s)
            pltpu.async_copy(rows[s], acc.at[didx[s]],
                             ssem[s], add=True).wait()
        plsc.subcore_barrier()
        pltpu.sync_copy(acc.at[pl.ds(rbase, rpt)],
                        out_hbm.at[cid].at[pl.ds(rbase, rpt)])

        @pl.when(sid < extras)
        def _():
            pltpu.sync_copy(acc.at[pl.ds(xbase, 8)],
                            out_hbm.at[cid].at[pl.ds(xbase, 8)])

    return k(x, src3, dst3, zeros)


def _hp_dot(a, b):
    return jnp.dot(a, b, preferred_element_type=jnp.float32,
                   precision=lax.Precision.HIGHEST)


def _bn_mlp(a0, a1, w1, b1, gamma, beta, w2, b2):
    n, d = a0.shape
    inv_n = 1.0 / n

    def body(a0r, a1r, w1r, b1r, gr, br, w2r, b2r, hr):
        xin = a0r[...] + a1r[...]
        hpre = _hp_dot(xin, w1r[...]) + b1r[...]
        m = jnp.mean(hpre, axis=0, keepdims=True)
        c = hpre - m
        v = jnp.sum(c * c, axis=0, keepdims=True) * inv_n
        h1 = jnp.maximum(c * (gr[...] * lax.rsqrt(v + _EPS)) + br[...], 0.0)
        hr[...] = jnp.maximum(_hp_dot(h1, w2r[...]) + b2r[...], 0.0)

    full = lambda r, c: pl.BlockSpec((r, c), lambda: (0, 0))
    return pl.pallas_call(
        body,
        in_specs=[full(n, d), full(n, d), full(d, d), full(1, d),
                  full(1, d), full(1, d), full(d, d), full(1, d)],
        out_specs=full(n, d),
        out_shape=jax.ShapeDtypeStruct((n, d), jnp.float32),
    )(a0, a1, w1, b1, gamma, beta, w2, b2)


def _bn_mlp_pool(a0, a1, w1, b1, gamma, beta, w2, b2, bid, g):
    n, d = a0.shape
    inv_n = 1.0 / n

    def body(a0r, a1r, w1r, b1r, gr, br, w2r, b2r, bidr, outr):
        xin = a0r[...] + a1r[...]
        hpre = _hp_dot(xin, w1r[...]) + b1r[...]
        m = jnp.mean(hpre, axis=0, keepdims=True)
        c = hpre - m
        v = jnp.sum(c * c, axis=0, keepdims=True) * inv_n
        h1 = jnp.maximum(c * (gr[...] * lax.rsqrt(v + _EPS)) + br[...], 0.0)
        h2 = jnp.maximum(_hp_dot(h1, w2r[...]) + b2r[...], 0.0)
        ids = bidr[...]
        gcol = lax.broadcasted_iota(jnp.int32, (n, g), 1).astype(jnp.float32)
        onehot = (ids == gcol).astype(jnp.float32)
        sums = lax.dot_general(onehot, h2, (((0,), (0,)), ((), ())),
                               preferred_element_type=jnp.float32,
                               precision=lax.Precision.HIGHEST)
        cnt = jnp.sum(onehot, axis=0)[:, None]
        outr[...] = sums / jnp.maximum(cnt, 1.0)

    full = lambda r, c: pl.BlockSpec((r, c), lambda: (0, 0))
    return pl.pallas_call(
        body,
        in_specs=[full(n, d), full(n, d), full(d, d), full(1, d),
                  full(1, d), full(1, d), full(d, d), full(1, d),
                  full(n, 1)],
        out_specs=full(g, d),
        out_shape=jax.ShapeDtypeStruct((g, d), jnp.float32),
    )(a0, a1, w1, b1, gamma, beta, w2, b2, bid)


def kernel(x, edge_index, batch, W1_0, b1_0, gamma_0, beta_0, W2_0, b2_0,
           W1_1, b1_1, gamma_1, beta_1, W2_1, b2_1):
    n, d = x.shape
    g = 64
    src3 = edge_index[0].reshape(NW, -1, _CHUNK)
    dst3 = edge_index[1].reshape(NW, -1, _CHUNK)
    z = jnp.zeros_like(x)
    bid = batch.astype(jnp.float32).reshape(n, 1)

    agg = _sc_segment_sum(x, src3, dst3, z)
    h = _bn_mlp(agg[0], agg[1], W1_0, b1_0.reshape(1, d),
                gamma_0.reshape(1, d), beta_0.reshape(1, d),
                W2_0, b2_0.reshape(1, d))

    agg2 = _sc_segment_sum(h, src3, dst3, z)
    out = _bn_mlp_pool(agg2[0], agg2[1], W1_1, b1_1.reshape(1, d),
                       gamma_1.reshape(1, d), beta_1.reshape(1, d),
                       W2_1, b2_1.reshape(1, d), bid, g)
    return out


# speedup vs baseline: 8.6273x; 1.0552x over previous
import functools

import jax
import jax.numpy as jnp
from jax import lax
from jax.experimental import pallas as pl
from jax.experimental.pallas import tpu as pltpu
from jax.experimental.pallas import tpu_sc as plsc

NC = 2
NS = 16
NW = NC * NS

_EPS = 1e-5


_CHUNK = 50
_SLOTS = 5


def _sc_segment_sum(x, src3, dst3, zeros):
    n, d = x.shape
    _, steps, chunk = dst3.shape
    units = n // 8
    upt = units // NS
    rpt = upt * 8
    extras = units - upt * NS
    mesh = plsc.VectorSubcoreMesh(core_axis_name="c", subcore_axis_name="s")

    @functools.partial(
        pl.kernel,
        out_type=jax.ShapeDtypeStruct((NC, n, d), jnp.float32),
        mesh=mesh,
        scratch_types=[
            [pltpu.VMEM((chunk,), jnp.int32)] * _SLOTS,
            [pltpu.VMEM((chunk,), jnp.int32)] * _SLOTS,
            [pltpu.VMEM((chunk, d), jnp.float32)] * _SLOTS,
            pltpu.MemorySpace.VMEM_SHARED((n, d), jnp.float32),
            [pltpu.SemaphoreType.DMA] * _SLOTS,
            [pltpu.SemaphoreType.DMA] * _SLOTS,
            [pltpu.SemaphoreType.DMA] * _SLOTS,
            [pltpu.SemaphoreType.DMA] * _SLOTS,
        ],
    )
    def k(x_hbm, src_hbm, dst_hbm, z_hbm, out_hbm, sidx, didx, rows, acc,
          isem, dsem, gsem, ssem):
        cid = lax.axis_index("c")
        sid = lax.axis_index("s")
        wid = sid * NC + cid
        rbase = pl.multiple_of(sid * rpt, 8)
        xbase = pl.multiple_of(upt * NS * 8 + sid * 8, 8)

        def init_from(ref):
            pltpu.sync_copy(ref.at[pl.ds(rbase, rpt)], acc.at[pl.ds(rbase, rpt)])

            @pl.when(sid < extras)
            def _():
                pltpu.sync_copy(ref.at[pl.ds(xbase, 8)], acc.at[pl.ds(xbase, 8)])

        @pl.when(cid == 0)
        def _():
            init_from(x_hbm)

        @pl.when(cid != 0)
        def _():
            init_from(z_hbm)

        def start_idx(j, slot):
            pltpu.async_copy(src_hbm.at[wid].at[j], sidx[slot], isem[slot])
            pltpu.async_copy(dst_hbm.at[wid].at[j], didx[slot], dsem[slot])

        def wait_idx(slot):
            pltpu.make_async_copy(src_hbm.at[wid].at[0], sidx[slot],
                                  isem[slot]).wait()
            pltpu.make_async_copy(dst_hbm.at[wid].at[0], didx[slot],
                                  dsem[slot]).wait()

        def wait_gather(slot):
            pltpu.make_async_copy(x_hbm.at[sidx[0]], rows[slot],
                                  gsem[slot]).wait()

        for s in range(_SLOTS):
            start_idx(s, s)

        def body(i, carry):
            j = i * _SLOTS
            scat = []
            for s in range(_SLOTS):
                wait_idx(s)
                pltpu.async_copy(x_hbm.at[sidx[s]], rows[s], gsem[s])
            for s in range(_SLOTS):
                wait_gather(s)
                scat.append(pltpu.async_copy(rows[s], acc.at[didx[s]],
                                             ssem[s], add=True))
            for s in range(_SLOTS):
                scat[s].wait()
                start_idx(j + s + _SLOTS, s)
            return carry

        lax.fori_loop(0, steps // _SLOTS - 1, body, 0)
        for s in range(_SLOTS):
            wait_idx(s)
            pltpu.async_copy(x_hbm.at[sidx[s]], rows[s], gsem[s])
        for s in range(_SLOTS):
            wait_gather(s)
            pltpu.async_copy(rows[s], acc.at[didx[s]],
                             ssem[s], add=True).wait()
        plsc.subcore_barrier()
        pltpu.sync_copy(acc.at[pl.ds(rbase, rpt)],
                        out_hbm.at[cid].at[pl.ds(rbase, rpt)])

        @pl.when(sid < extras)
        def _():
            pltpu.sync_copy(acc.at[pl.ds(xbase, 8)],
                            out_hbm.at[cid].at[pl.ds(xbase, 8)])

    return k(x, src3, dst3, zeros)


def _hp_dot(a, b):
    return jnp.dot(a, b, preferred_element_type=jnp.float32,
                   precision=lax.Precision.HIGHEST)


def _bn_mlp(agg, w1, b1, gamma, beta, w2, b2):
    _, n, d = agg.shape
    inv_n = 1.0 / n

    def body(ar, w1r, b1r, gr, br, w2r, b2r, hr):
        xin = ar[0] + ar[1]
        hpre = _hp_dot(xin, w1r[...]) + b1r[...]
        m = jnp.mean(hpre, axis=0, keepdims=True)
        c = hpre - m
        v = jnp.sum(c * c, axis=0, keepdims=True) * inv_n
        h1 = jnp.maximum(c * (gr[...] * lax.rsqrt(v + _EPS)) + br[...], 0.0)
        hr[...] = jnp.maximum(_hp_dot(h1, w2r[...]) + b2r[...], 0.0)

    full = lambda r, c: pl.BlockSpec((r, c), lambda: (0, 0))
    return pl.pallas_call(
        body,
        in_specs=[pl.BlockSpec((2, n, d), lambda: (0, 0, 0)),
                  full(d, d), full(1, d),
                  full(1, d), full(1, d), full(d, d), full(1, d)],
        out_specs=full(n, d),
        out_shape=jax.ShapeDtypeStruct((n, d), jnp.float32),
    )(agg, w1, b1, gamma, beta, w2, b2)


def _bn_mlp_pool(agg, w1, b1, gamma, beta, w2, b2, bid, g):
    _, n, d = agg.shape
    inv_n = 1.0 / n

    def body(ar, w1r, b1r, gr, br, w2r, b2r, bidr, outr):
        xin = ar[0] + ar[1]
        hpre = _hp_dot(xin, w1r[...]) + b1r[...]
        m = jnp.mean(hpre, axis=0, keepdims=True)
        c = hpre - m
        v = jnp.sum(c * c, axis=0, keepdims=True) * inv_n
        h1 = jnp.maximum(c * (gr[...] * lax.rsqrt(v + _EPS)) + br[...], 0.0)
        h2 = jnp.maximum(_hp_dot(h1, w2r[...]) + b2r[...], 0.0)
        ids = bidr[...]
        gcol = lax.broadcasted_iota(jnp.int32, (n, g), 1).astype(jnp.float32)
        onehot = (ids == gcol).astype(jnp.float32)
        sums = lax.dot_general(onehot, h2, (((0,), (0,)), ((), ())),
                               preferred_element_type=jnp.float32,
                               precision=lax.Precision.HIGHEST)
        cnt = jnp.sum(onehot, axis=0)[:, None]
        outr[...] = sums / jnp.maximum(cnt, 1.0)

    full = lambda r, c: pl.BlockSpec((r, c), lambda: (0, 0))
    return pl.pallas_call(
        body,
        in_specs=[pl.BlockSpec((2, n, d), lambda: (0, 0, 0)),
                  full(d, d), full(1, d),
                  full(1, d), full(1, d), full(d, d), full(1, d),
                  full(n, 1)],
        out_specs=full(g, d),
        out_shape=jax.ShapeDtypeStruct((g, d), jnp.float32),
    )(agg, w1, b1, gamma, beta, w2, b2, bid)


def kernel(x, edge_index, batch, W1_0, b1_0, gamma_0, beta_0, W2_0, b2_0,
           W1_1, b1_1, gamma_1, beta_1, W2_1, b2_1):
    n, d = x.shape
    g = 64
    src3 = edge_index[0].reshape(NW, -1, _CHUNK)
    dst3 = edge_index[1].reshape(NW, -1, _CHUNK)
    z = jnp.zeros_like(x)
    bid = batch.astype(jnp.float32).reshape(n, 1)

    agg = _sc_segment_sum(x, src3, dst3, z)
    h = _bn_mlp(agg, W1_0, b1_0.reshape(1, d),
                gamma_0.reshape(1, d), beta_0.reshape(1, d),
                W2_0, b2_0.reshape(1, d))

    agg2 = _sc_segment_sum(h, src3, dst3, z)
    out = _bn_mlp_pool(agg2, W1_1, b1_1.reshape(1, d),
                       gamma_1.reshape(1, d), beta_1.reshape(1, d),
                       W2_1, b2_1.reshape(1, d), bid, g)
    return out
